# Initial kernel scaffold; baseline (speedup 1.0000x reference)
import jax
import jax.numpy as jnp
from jax.experimental import pallas as pl


def kernel(shape_xyz, skel_xyz, skel_nori):
    raise NotImplementedError("write your pallas kernel here")


# speedup vs baseline: 18.8698x; 18.8698x over previous
import jax
import jax.numpy as jnp
from jax import lax
from jax.experimental import pallas as pl
from jax.experimental.pallas import tpu as pltpu

_B, _N, _M = 8, 4096, 256
_NCH = 256
_NB = _N // _NCH
_BIGF = 1e30
_BIGI = 1 << 30


def _body(shape_ref, skelT_ref, noriT_ref, out_ref, cda, nra, m1, d1, m2, d2):
    b = pl.program_id(0)
    nb = pl.program_id(1)

    blk = shape_ref[0]
    px, py, pz = blk[:, 0:1], blk[:, 1:2], blk[:, 2:3]
    nx, ny, nz = blk[:, 3:4], blk[:, 4:5], blk[:, 5:6]
    sk = skelT_ref[0]
    sx, sy, sz = sk[0:1, :], sk[1:2, :], sk[2:3, :]
    no = noriT_ref[0]
    ox, oy, oz = no[0:1, :], no[1:2, :], no[2:3, :]

    dxx = px - sx
    dyy = py - sy
    dzz = pz - sz
    d2m = dxx * dxx + dyy * dyy + dzz * dzz
    dots = nx * ox + ny * oy + nz * oz

    c1 = jnp.min(d2m, axis=1, keepdims=True)
    cd_part = jnp.sum(jnp.sqrt(c1 + 1e-12), keepdims=True).reshape(1, 1)

    ri = lax.broadcasted_iota(jnp.int32, (_NCH, _M), 0)
    bm1 = jnp.min(d2m, axis=0, keepdims=True)
    bi1 = jnp.min(jnp.where(d2m == bm1, ri, _BIGI), axis=0, keepdims=True)
    sel1 = ri == bi1
    bd1 = jnp.sum(jnp.where(sel1, dots, 0.0), axis=0, keepdims=True)
    mk = jnp.where(sel1, _BIGF, d2m)
    bm2 = jnp.min(mk, axis=0, keepdims=True)
    bi2 = jnp.min(jnp.where(mk == bm2, ri, _BIGI), axis=0, keepdims=True)
    bd2 = jnp.sum(jnp.where(ri == bi2, dots, 0.0), axis=0, keepdims=True)

    @pl.when(nb == 0)
    def _init():
        m1[...] = jnp.full((1, _M), _BIGF)
        m2[...] = jnp.full((1, _M), _BIGF)
        d1[...] = jnp.zeros((1, _M), jnp.float32)
        d2[...] = jnp.zeros((1, _M), jnp.float32)

    @pl.when((b == 0) & (nb == 0))
    def _init_acc():
        cda[...] = jnp.zeros((1, 1), jnp.float32)
        nra[...] = jnp.zeros((1, 1), jnp.float32)

    rm1, rd1, rm2, rd2 = m1[...], d1[...], m2[...], d2[...]
    c1lt = bm1 < rm1
    nm1 = jnp.where(c1lt, bm1, rm1)
    nv1 = jnp.where(c1lt, bd1, rd1)
    cm = jnp.where(c1lt, rm1, rm2)
    cv = jnp.where(c1lt, rd1, rd2)
    cbm = jnp.where(c1lt, bm2, bm1)
    cbv = jnp.where(c1lt, bd2, bd1)
    c2lt = cbm < cm
    nm2 = jnp.where(c2lt, cbm, cm)
    nv2 = jnp.where(c2lt, cbv, cv)
    m1[...] = nm1
    d1[...] = nv1
    m2[...] = nm2
    d2[...] = nv2

    cda[...] = cda[...] + cd_part

    @pl.when(nb == _NB - 1)
    def _fin_batch():
        cd2v = jnp.sum(jnp.sqrt(m1[...] + 1e-12), keepdims=True).reshape(1, 1)
        nrm = jnp.sum(jnp.abs(d1[...]) + jnp.abs(d2[...]),
                      keepdims=True).reshape(1, 1) * 0.5
        cda[...] = cda[...] + cd2v
        nra[...] = nra[...] + nrm

    @pl.when((b == _B - 1) & (nb == _NB - 1))
    def _emit():
        out_ref[...] = cda[...] * 1e-4 + 0.001 * (nra[...] / _B)


def kernel(shape_xyz, skel_xyz, skel_nori):
    skelT = jnp.transpose(skel_xyz, (0, 2, 1))
    noriT = jnp.transpose(skel_nori, (0, 2, 1))
    out = pl.pallas_call(
        _body,
        grid=(_B, _NB),
        in_specs=[
            pl.BlockSpec((1, _NCH, 6), lambda b, nb: (b, nb, 0)),
            pl.BlockSpec((1, 3, _M), lambda b, nb: (b, 0, 0)),
            pl.BlockSpec((1, 3, _M), lambda b, nb: (b, 0, 0)),
        ],
        out_specs=pl.BlockSpec((1, 1), lambda b, nb: (0, 0)),
        out_shape=jax.ShapeDtypeStruct((1, 1), jnp.float32),
        scratch_shapes=[
            pltpu.VMEM((1, 1), jnp.float32),
            pltpu.VMEM((1, 1), jnp.float32),
            pltpu.VMEM((1, _M), jnp.float32),
            pltpu.VMEM((1, _M), jnp.float32),
            pltpu.VMEM((1, _M), jnp.float32),
            pltpu.VMEM((1, _M), jnp.float32),
        ],
    )(shape_xyz, skelT, noriT)
    return out[0, 0]
